# baseline (speedup 1.0000x reference)
import jax
import jax.numpy as jnp
from jax.experimental import pallas as pl
from jax.experimental.pallas import tpu as pltpu


def _body(idx_hbm, upd_hbm, par_hbm, out_hbm, idx_s, s0, s1, s2):
    c0 = pltpu.make_async_copy(idx_hbm, idx_s, s0)
    c1 = pltpu.make_async_copy(par_hbm, out_hbm, s1)
    c0.start()
    c1.start()
    c0.wait()
    c1.wait()
    i = idx_s[0]
    c2 = pltpu.make_async_copy(upd_hbm, out_hbm.at[pl.ds(i, 1)], s2)
    c2.start()
    c2.wait()


def kernel(update, index, params):
    out = pl.pallas_call(
        _body,
        out_shape=jax.ShapeDtypeStruct((2, 10), jnp.float32),
        in_specs=[pl.BlockSpec(memory_space=pl.ANY)] * 3,
        out_specs=pl.BlockSpec(memory_space=pl.ANY),
        scratch_shapes=[pltpu.SMEM((1,), jnp.int32)]
        + [pltpu.SemaphoreType.DMA] * 3,
    )(index, update.reshape(1, 10), params.reshape(2, 10))
    return out.reshape(2, 1, 10)


# speedup vs baseline: 1.3905x; 1.3905x over previous
import jax
import jax.numpy as jnp
from jax import lax
from jax.experimental import pallas as pl
from jax.experimental.pallas import tpu as pltpu


def _tc_body(idx_ref, upd_ref, params_ref, out_ref):
    i = idx_ref[0]
    rows = lax.broadcasted_iota(jnp.int32, (2, 10), 0)
    out_ref[...] = jnp.where(rows == i, upd_ref[...], params_ref[...])


def kernel(update, index, params):
    out = pl.pallas_call(
        _tc_body,
        out_shape=jax.ShapeDtypeStruct((2, 10), jnp.float32),
        in_specs=[
            pl.BlockSpec(memory_space=pltpu.SMEM),
            pl.BlockSpec(memory_space=pltpu.VMEM),
            pl.BlockSpec(memory_space=pltpu.VMEM),
        ],
        out_specs=pl.BlockSpec(memory_space=pltpu.VMEM),
    )(index, update.reshape(1, 10), params.reshape(2, 10))
    return out.reshape(2, 1, 10)
